# Initial kernel scaffold; baseline (speedup 1.0000x reference)
import jax
import jax.numpy as jnp
from jax.experimental import pallas as pl


def kernel(xv, xc, adj_pos, adj_neg, meta_lit_idx, meta_lit_val, meta_cls_idx, meta_cls_val, W_lit, W_cls, W_cv_pos, W_cv_neg, W_vc_pos, W_vc_neg, gamma, beta):
    raise NotImplementedError("write your pallas kernel here")


# speedup vs baseline: 3.1362x; 3.1362x over previous
import dataclasses
import functools

import jax
import jax.numpy as jnp
from jax import lax
from jax.experimental import pallas as pl
from jax.experimental.pallas import tpu as pltpu
from jax.experimental.pallas import tpu_sc as plsc

N = 10000
D = 128
E = 160000
EM = 320000
L = 2
NS = 16
CHUNK = 128
ZROWS = 200

_mesh = plsc.VectorSubcoreMesh(core_axis_name="c", subcore_axis_name="s")

_cp = pltpu.CompilerParams()
if "needs_layout_passes" in pltpu.CompilerParams.__dataclass_fields__:
    _cp = dataclasses.replace(_cp, needs_layout_passes=False)


@functools.partial(
    pl.kernel,
    out_type=jax.ShapeDtypeStruct((12, N, D), jnp.float32),
    mesh=_mesh,
    scratch_types=[
        pltpu.VMEM((CHUNK,), jnp.int32),
        pltpu.VMEM((CHUNK,), jnp.int32),
        pltpu.VMEM((CHUNK,), jnp.float32),
        pltpu.VMEM((CHUNK, D), jnp.float32),
        pltpu.VMEM((ZROWS, D), jnp.float32),
        pltpu.VMEM_SHARED((N, D), jnp.float32),
        pltpu.SemaphoreType.DMA,
    ],
    compiler_params=_cp,
)
def _sc_aggregate(xv, xc, mli, mlv, mci, mcv, adjp, adjn, out,
                  idx_src, idx_dst, val_v, rows, zbuf, acc, sem):
    core = lax.axis_index("c")
    sid = lax.axis_index("s")

    def zero_own():
        @pl.when(sid < 10)
        def _():
            @pl.loop(0, 5)
            def _(k):
                pltpu.sync_copy(zbuf, acc.at[pl.ds(sid * 1000 + k * ZROWS, ZROWS)])

    def drain_own(slot):
        @pl.when(sid < 10)
        def _():
            @pl.loop(0, 5)
            def _(k):
                b = sid * 1000 + k * ZROWS
                pltpu.sync_copy(acc.at[pl.ds(b, ZROWS)], out.at[slot, pl.ds(b, ZROWS)])

    def do_agg(table, src_hbm, dst_hbm, val_hbm, slot, n_edges):
        nchunks = n_edges // CHUNK
        my_chunks = nchunks // NS + jnp.where(sid < nchunks % NS, 1, 0)

        @pl.loop(0, my_chunks)
        def _(c):
            b = (c * NS + sid) * CHUNK
            pltpu.sync_copy(src_hbm.at[pl.ds(b, CHUNK)], idx_src)
            pltpu.sync_copy(dst_hbm.at[pl.ds(b, CHUNK)], idx_dst)
            pltpu.async_copy(table.at[idx_src], rows, sem).wait()
            if val_hbm is not None:
                pltpu.sync_copy(val_hbm.at[pl.ds(b, CHUNK)], val_v)

                @pl.loop(0, CHUNK)
                def _(i):
                    vv = plsc.load_gather(val_v, [jnp.full((16,), i, jnp.int32)])
                    for j in range(0, D, 16):
                        rows[i, pl.ds(j, 16)] = rows[i, pl.ds(j, 16)] * vv
            pltpu.sync_copy(rows, acc.at[idx_dst], add=True)

        plsc.subcore_barrier()
        drain_own(slot)
        zero_own()
        plsc.subcore_barrier()

    @pl.loop(0, ZROWS)
    def _(i):
        @pl.loop(0, D, step=16)
        def _(j):
            zbuf[i, pl.ds(j, 16)] = jnp.zeros((16,), jnp.float32)

    zero_own()
    plsc.subcore_barrier()

    @pl.when(core == 0)
    def _():
        for p in range(4):
            do_agg(xv, mli.at[p, 1], mli.at[p, 0], mlv.at[p], p, EM)
        do_agg(xc, adjp.at[0], adjp.at[1], None, 4, E)
        do_agg(xc, adjn.at[0], adjn.at[1], None, 5, E)

    @pl.when(core == 1)
    def _():
        for p in range(4):
            do_agg(xc, mci.at[p, 1], mci.at[p, 0], mcv.at[p], 6 + p, EM)
        do_agg(xv, adjp.at[1], adjp.at[0], None, 10, E)
        do_agg(xv, adjn.at[1], adjn.at[0], None, 11, E)


BLK = 1000


def _tc_body(final, xv_ref, xc_ref, aggs_ref, w_ref, g_ref, b_ref,
             hv_ref, hc_ref):
    def side(x_ref, k0, out_ref):
        h = x_ref[...]
        for k in range(k0, k0 + 6):
            h = h + jnp.maximum(
                jnp.dot(aggs_ref[k], w_ref[k],
                        preferred_element_type=jnp.float32), 0.0)
        if final:
            mu = jnp.mean(h, axis=-1, keepdims=True)
            var = jnp.mean((h - mu) ** 2, axis=-1, keepdims=True)
            h = (h - mu) * lax.rsqrt(var + 1e-5) * g_ref[...] + b_ref[...]
        out_ref[...] = h

    side(xv_ref, 0, hv_ref)
    side(xc_ref, 6, hc_ref)


def _tc_update(xv, xc, aggs, w12, gamma2, beta2, final):
    return pl.pallas_call(
        functools.partial(_tc_body, final),
        grid=(N // BLK,),
        in_specs=[
            pl.BlockSpec((BLK, D), lambda i: (i, 0)),
            pl.BlockSpec((BLK, D), lambda i: (i, 0)),
            pl.BlockSpec((12, BLK, D), lambda i: (0, i, 0)),
            pl.BlockSpec((12, D, D), lambda i: (0, 0, 0)),
            pl.BlockSpec((1, D), lambda i: (0, 0)),
            pl.BlockSpec((1, D), lambda i: (0, 0)),
        ],
        out_specs=[
            pl.BlockSpec((BLK, D), lambda i: (i, 0)),
            pl.BlockSpec((BLK, D), lambda i: (i, 0)),
        ],
        out_shape=[jax.ShapeDtypeStruct((N, D), jnp.float32)] * 2,
    )(xv, xc, aggs, w12, gamma2, beta2)


def kernel(xv, xc, adj_pos, adj_neg, meta_lit_idx, meta_lit_val,
           meta_cls_idx, meta_cls_val, W_lit, W_cls, W_cv_pos, W_cv_neg,
           W_vc_pos, W_vc_neg, gamma, beta):
    adjp = adj_pos.astype(jnp.int32)
    adjn = adj_neg.astype(jnp.int32)
    mli = meta_lit_idx.astype(jnp.int32)
    mci = meta_cls_idx.astype(jnp.int32)
    gamma2 = gamma.reshape(1, D)
    beta2 = beta.reshape(1, D)
    for l in range(L):
        aggs = _sc_aggregate(xv, xc, mli, meta_lit_val, mci, meta_cls_val,
                             adjp, adjn)
        w12 = jnp.concatenate([
            W_lit[l], W_cv_pos[l][None], W_cv_neg[l][None],
            W_cls[l], W_vc_pos[l][None], W_vc_neg[l][None],
        ], axis=0)
        xv, xc = _tc_update(xv, xc, aggs, w12, gamma2, beta2, final=(l == L - 1))
    return (xv, xc)


# speedup vs baseline: 4.3984x; 1.4025x over previous
import dataclasses
import functools

import jax
import jax.numpy as jnp
from jax import lax
from jax.experimental import pallas as pl
from jax.experimental.pallas import tpu as pltpu
from jax.experimental.pallas import tpu_sc as plsc

N = 10000
D = 128
E = 160000
EM = 320000
L = 2
NS = 16
CHUNK = 128

_mesh = plsc.VectorSubcoreMesh(core_axis_name="c", subcore_axis_name="s")

_cp = pltpu.CompilerParams()
if "needs_layout_passes" in pltpu.CompilerParams.__dataclass_fields__:
    _cp = dataclasses.replace(_cp, needs_layout_passes=False)


@functools.partial(
    pl.kernel,
    out_type=jax.ShapeDtypeStruct((12, N, D), jnp.float32),
    mesh=_mesh,
    scratch_types=[
        pltpu.VMEM((2, CHUNK), jnp.int32),
        pltpu.VMEM((2, CHUNK), jnp.int32),
        pltpu.VMEM((2, CHUNK), jnp.float32),
        pltpu.VMEM((2, CHUNK, D), jnp.float32),
        pltpu.VMEM_SHARED((N, D), jnp.float32),
        pltpu.SemaphoreType.DMA,
        pltpu.SemaphoreType.DMA,
    ],
    compiler_params=_cp,
)
def _sc_aggregate(xv, xc, mli, mlv, mci, mcv, adjp, adjn, zeros_hbm, out,
                  idx_src, idx_dst, val_v, rows, acc, sem0, sem1):
    core = lax.axis_index("c")
    sid = lax.axis_index("s")

    def zero_own():
        @pl.when(sid < 10)
        def _():
            pltpu.sync_copy(zeros_hbm, acc.at[pl.ds(sid * 1000, 1000)])

    def drain_own(slot):
        @pl.when(sid < 10)
        def _():
            b = sid * 1000
            pltpu.sync_copy(acc.at[pl.ds(b, 1000)], out.at[slot, pl.ds(b, 1000)])

    def do_agg(table, src_hbm, dst_hbm, val_hbm, slot, n_edges):
        nchunks = n_edges // CHUNK
        my = nchunks // NS + jnp.where(sid < nchunks % NS, 1, 0)
        sems = (sem0, sem1)

        def fetch(c, p):
            b = (c * NS + sid) * CHUNK
            pltpu.sync_copy(src_hbm.at[pl.ds(b, CHUNK)], idx_src.at[p])
            pltpu.sync_copy(dst_hbm.at[pl.ds(b, CHUNK)], idx_dst.at[p])
            if val_hbm is not None:
                pltpu.sync_copy(val_hbm.at[pl.ds(b, CHUNK)], val_v.at[p])
            pltpu.async_copy(table.at[idx_src.at[p]], rows.at[p], sems[p])

        def process(p):
            pltpu.make_async_copy(table.at[idx_src.at[p]], rows.at[p],
                                  sems[p]).wait()
            if val_hbm is not None:
                @pl.loop(0, CHUNK)
                def _(i):
                    vv = plsc.load_gather(val_v.at[p],
                                          [jnp.full((16,), i, jnp.int32)])
                    for j in range(0, D, 16):
                        rows[p, i, pl.ds(j, 16)] = rows[p, i, pl.ds(j, 16)] * vv
            pltpu.sync_copy(rows.at[p], acc.at[idx_dst.at[p]], add=True)

        @pl.when(my > 0)
        def _():
            fetch(0, 0)

            @pl.loop(0, (my + 1) // 2)
            def _(k):
                c = k * 2

                @pl.when(c + 1 < my)
                def _():
                    fetch(c + 1, 1)
                process(0)

                @pl.when(c + 1 < my)
                def _():
                    @pl.when(c + 2 < my)
                    def _():
                        fetch(c + 2, 0)
                    process(1)

        plsc.subcore_barrier()
        drain_own(slot)
        zero_own()
        plsc.subcore_barrier()

    zero_own()
    plsc.subcore_barrier()

    @pl.when(core == 0)
    def _():
        for p in range(4):
            do_agg(xv, mli.at[p, 1], mli.at[p, 0], mlv.at[p], p, EM)
        do_agg(xc, adjp.at[0], adjp.at[1], None, 4, E)
        do_agg(xc, adjn.at[0], adjn.at[1], None, 5, E)

    @pl.when(core == 1)
    def _():
        for p in range(4):
            do_agg(xc, mci.at[p, 1], mci.at[p, 0], mcv.at[p], 6 + p, EM)
        do_agg(xv, adjp.at[1], adjp.at[0], None, 10, E)
        do_agg(xv, adjn.at[1], adjn.at[0], None, 11, E)


BLK = 1000


def _tc_body(final, xv_ref, xc_ref, aggs_ref, w_ref, g_ref, b_ref,
             hv_ref, hc_ref):
    def side(x_ref, k0, out_ref):
        h = x_ref[...]
        for k in range(k0, k0 + 6):
            h = h + jnp.maximum(
                jnp.dot(aggs_ref[k], w_ref[k],
                        preferred_element_type=jnp.float32), 0.0)
        if final:
            mu = jnp.mean(h, axis=-1, keepdims=True)
            var = jnp.mean((h - mu) ** 2, axis=-1, keepdims=True)
            h = (h - mu) * lax.rsqrt(var + 1e-5) * g_ref[...] + b_ref[...]
        out_ref[...] = h

    side(xv_ref, 0, hv_ref)
    side(xc_ref, 6, hc_ref)


def _tc_update(xv, xc, aggs, w12, gamma2, beta2, final):
    return pl.pallas_call(
        functools.partial(_tc_body, final),
        grid=(N // BLK,),
        in_specs=[
            pl.BlockSpec((BLK, D), lambda i: (i, 0)),
            pl.BlockSpec((BLK, D), lambda i: (i, 0)),
            pl.BlockSpec((12, BLK, D), lambda i: (0, i, 0)),
            pl.BlockSpec((12, D, D), lambda i: (0, 0, 0)),
            pl.BlockSpec((1, D), lambda i: (0, 0)),
            pl.BlockSpec((1, D), lambda i: (0, 0)),
        ],
        out_specs=[
            pl.BlockSpec((BLK, D), lambda i: (i, 0)),
            pl.BlockSpec((BLK, D), lambda i: (i, 0)),
        ],
        out_shape=[jax.ShapeDtypeStruct((N, D), jnp.float32)] * 2,
    )(xv, xc, aggs, w12, gamma2, beta2)


def kernel(xv, xc, adj_pos, adj_neg, meta_lit_idx, meta_lit_val,
           meta_cls_idx, meta_cls_val, W_lit, W_cls, W_cv_pos, W_cv_neg,
           W_vc_pos, W_vc_neg, gamma, beta):
    adjp = adj_pos.astype(jnp.int32)
    adjn = adj_neg.astype(jnp.int32)
    mli = meta_lit_idx.astype(jnp.int32)
    mci = meta_cls_idx.astype(jnp.int32)
    gamma2 = gamma.reshape(1, D)
    beta2 = beta.reshape(1, D)
    zeros_hbm = jnp.zeros((1000, D), jnp.float32)
    for l in range(L):
        aggs = _sc_aggregate(xv, xc, mli, meta_lit_val, mci, meta_cls_val,
                             adjp, adjn, zeros_hbm)
        w12 = jnp.concatenate([
            W_lit[l], W_cv_pos[l][None], W_cv_neg[l][None],
            W_cls[l], W_vc_pos[l][None], W_vc_neg[l][None],
        ], axis=0)
        xv, xc = _tc_update(xv, xc, aggs, w12, gamma2, beta2, final=(l == L - 1))
    return (xv, xc)


# speedup vs baseline: 7.1259x; 1.6201x over previous
import dataclasses
import functools

import jax
import jax.numpy as jnp
from jax import lax
from jax.experimental import pallas as pl
from jax.experimental.pallas import tpu as pltpu
from jax.experimental.pallas import tpu_sc as plsc

N = 10000
D = 128
E = 160000
EM = 320000
L = 2
NS = 16
CHUNK = 128

_mesh = plsc.VectorSubcoreMesh(core_axis_name="c", subcore_axis_name="s")

_cp = pltpu.CompilerParams()
if "needs_layout_passes" in pltpu.CompilerParams.__dataclass_fields__:
    _cp = dataclasses.replace(_cp, needs_layout_passes=False)


@functools.partial(
    pl.kernel,
    out_type=jax.ShapeDtypeStruct((12, N, D), jnp.float32),
    mesh=_mesh,
    scratch_types=[
        pltpu.VMEM((3, CHUNK), jnp.int32),
        pltpu.VMEM((3, CHUNK), jnp.int32),
        pltpu.VMEM((3, CHUNK), jnp.float32),
        pltpu.VMEM((3, CHUNK, D), jnp.float32),
        pltpu.VMEM_SHARED((N, D), jnp.float32),
    ] + [pltpu.SemaphoreType.DMA] * 9,
    compiler_params=_cp,
)
def _sc_aggregate(xv, xc, mli, mlv, mci, mcv, adjp, adjn, zeros_hbm, out,
                  idx_src, idx_dst, val_v, rows, acc, *sems9):
    isem = sems9[0:3]
    gsem = sems9[3:6]
    ssem = sems9[6:9]
    core = lax.axis_index("c")
    sid = lax.axis_index("s")

    def zero_own():
        @pl.when(sid < 10)
        def _():
            pltpu.sync_copy(zeros_hbm, acc.at[pl.ds(sid * 1000, 1000)])

    def drain_own(slot):
        @pl.when(sid < 10)
        def _():
            b = sid * 1000
            pltpu.sync_copy(acc.at[pl.ds(b, 1000)], out.at[slot, pl.ds(b, 1000)])

    def do_agg(table, src_hbm, dst_hbm, val_hbm, slot, n_edges):
        nchunks = n_edges // CHUNK
        my = nchunks // NS + jnp.where(sid < nchunks % NS, 1, 0)

        def idx_fetch(c, b):
            off = (c * NS + sid) * CHUNK
            pltpu.async_copy(src_hbm.at[pl.ds(off, CHUNK)], idx_src.at[b],
                             isem[b])
            pltpu.async_copy(dst_hbm.at[pl.ds(off, CHUNK)], idx_dst.at[b],
                             isem[b])
            if val_hbm is not None:
                pltpu.async_copy(val_hbm.at[pl.ds(off, CHUNK)], val_v.at[b],
                                 isem[b])

        def idx_wait(b):
            pltpu.make_async_copy(src_hbm.at[pl.ds(0, CHUNK)], idx_src.at[b],
                                  isem[b]).wait()
            pltpu.make_async_copy(dst_hbm.at[pl.ds(0, CHUNK)], idx_dst.at[b],
                                  isem[b]).wait()
            if val_hbm is not None:
                pltpu.make_async_copy(val_hbm.at[pl.ds(0, CHUNK)],
                                      val_v.at[b], isem[b]).wait()

        def gather_issue(b):
            pltpu.async_copy(table.at[idx_src.at[b]], rows.at[b], gsem[b])

        def gather_wait(b):
            pltpu.make_async_copy(table.at[idx_src.at[b]], rows.at[b],
                                  gsem[b]).wait()

        def scatter_issue(b):
            pltpu.async_copy(rows.at[b], acc.at[idx_dst.at[b]], ssem[b],
                             add=True)

        def scatter_wait(b):
            pltpu.make_async_copy(rows.at[b], acc.at[idx_dst.at[b]],
                                  ssem[b]).wait()

        def scale(b):
            if val_hbm is not None:
                @pl.loop(0, CHUNK)
                def _(i):
                    vv = plsc.load_gather(val_v.at[b],
                                          [jnp.full((16,), i, jnp.int32)])
                    for j in range(0, D, 16):
                        rows[b, i, pl.ds(j, 16)] = rows[b, i, pl.ds(j, 16)] * vv

        idx_fetch(0, 0)
        idx_fetch(1, 1)
        idx_wait(0)
        gather_issue(0)

        @pl.loop(0, (my + 2) // 3)
        def _(k):
            for r in range(3):
                c = k * 3 + r
                b0, b1, b2 = r % 3, (r + 1) % 3, (r + 2) % 3

                @pl.when(c < my)
                def _(c=c, b0=b0, b1=b1, b2=b2):
                    @pl.when(c >= 1)
                    def _():
                        scatter_wait(b2)

                    @pl.when(c + 2 < my)
                    def _():
                        idx_fetch(c + 2, b2)

                    @pl.when(c + 1 < my)
                    def _():
                        idx_wait(b1)
                        gather_issue(b1)
                    gather_wait(b0)
                    scale(b0)
                    scatter_issue(b0)

        for j in range(3):
            @pl.when((my - 1) % 3 == j)
            def _(j=j):
                scatter_wait(j)

        plsc.subcore_barrier()
        drain_own(slot)
        zero_own()
        plsc.subcore_barrier()

    zero_own()
    plsc.subcore_barrier()

    @pl.when(core == 0)
    def _():
        for p in range(4):
            do_agg(xv, mli.at[p, 1], mli.at[p, 0], mlv.at[p], p, EM)
        do_agg(xc, adjp.at[0], adjp.at[1], None, 4, E)
        do_agg(xc, adjn.at[0], adjn.at[1], None, 5, E)

    @pl.when(core == 1)
    def _():
        for p in range(4):
            do_agg(xc, mci.at[p, 1], mci.at[p, 0], mcv.at[p], 6 + p, EM)
        do_agg(xv, adjp.at[1], adjp.at[0], None, 10, E)
        do_agg(xv, adjn.at[1], adjn.at[0], None, 11, E)


BLK = 1000


def _tc_body(final, xv_ref, xc_ref, aggs_ref, w_ref, g_ref, b_ref,
             hv_ref, hc_ref):
    def side(x_ref, k0, out_ref):
        h = x_ref[...]
        for k in range(k0, k0 + 6):
            h = h + jnp.maximum(
                jnp.dot(aggs_ref[k], w_ref[k],
                        preferred_element_type=jnp.float32), 0.0)
        if final:
            mu = jnp.mean(h, axis=-1, keepdims=True)
            var = jnp.mean((h - mu) ** 2, axis=-1, keepdims=True)
            h = (h - mu) * lax.rsqrt(var + 1e-5) * g_ref[...] + b_ref[...]
        out_ref[...] = h

    side(xv_ref, 0, hv_ref)
    side(xc_ref, 6, hc_ref)


def _tc_update(xv, xc, aggs, w12, gamma2, beta2, final):
    return pl.pallas_call(
        functools.partial(_tc_body, final),
        grid=(N // BLK,),
        in_specs=[
            pl.BlockSpec((BLK, D), lambda i: (i, 0)),
            pl.BlockSpec((BLK, D), lambda i: (i, 0)),
            pl.BlockSpec((12, BLK, D), lambda i: (0, i, 0)),
            pl.BlockSpec((12, D, D), lambda i: (0, 0, 0)),
            pl.BlockSpec((1, D), lambda i: (0, 0)),
            pl.BlockSpec((1, D), lambda i: (0, 0)),
        ],
        out_specs=[
            pl.BlockSpec((BLK, D), lambda i: (i, 0)),
            pl.BlockSpec((BLK, D), lambda i: (i, 0)),
        ],
        out_shape=[jax.ShapeDtypeStruct((N, D), jnp.float32)] * 2,
    )(xv, xc, aggs, w12, gamma2, beta2)


def kernel(xv, xc, adj_pos, adj_neg, meta_lit_idx, meta_lit_val,
           meta_cls_idx, meta_cls_val, W_lit, W_cls, W_cv_pos, W_cv_neg,
           W_vc_pos, W_vc_neg, gamma, beta):
    adjp = adj_pos.astype(jnp.int32)
    adjn = adj_neg.astype(jnp.int32)
    mli = meta_lit_idx.astype(jnp.int32)
    mci = meta_cls_idx.astype(jnp.int32)
    gamma2 = gamma.reshape(1, D)
    beta2 = beta.reshape(1, D)
    zeros_hbm = jnp.zeros((1000, D), jnp.float32)
    for l in range(L):
        aggs = _sc_aggregate(xv, xc, mli, meta_lit_val, mci, meta_cls_val,
                             adjp, adjn, zeros_hbm)
        w12 = jnp.concatenate([
            W_lit[l], W_cv_pos[l][None], W_cv_neg[l][None],
            W_cls[l], W_vc_pos[l][None], W_vc_neg[l][None],
        ], axis=0)
        xv, xc = _tc_update(xv, xc, aggs, w12, gamma2, beta2, final=(l == L - 1))
    return (xv, xc)


# speedup vs baseline: 8.9937x; 1.2621x over previous
import dataclasses
import functools

import jax
import jax.numpy as jnp
from jax import lax
from jax.experimental import pallas as pl
from jax.experimental.pallas import tpu as pltpu
from jax.experimental.pallas import tpu_sc as plsc

N = 10000
D = 128
E = 160000
EM = 320000
L = 2
NS = 16
CHUNK = 128

_mesh = plsc.VectorSubcoreMesh(core_axis_name="c", subcore_axis_name="s")

_cp = pltpu.CompilerParams()
if "needs_layout_passes" in pltpu.CompilerParams.__dataclass_fields__:
    _cp = dataclasses.replace(_cp, needs_layout_passes=False)


@functools.partial(
    pl.kernel,
    out_type=jax.ShapeDtypeStruct((12, N, D), jnp.float32),
    mesh=_mesh,
    scratch_types=[
        pltpu.VMEM((3, CHUNK), jnp.int32),
        pltpu.VMEM((3, CHUNK), jnp.int32),
        pltpu.VMEM((3, CHUNK), jnp.float32),
        pltpu.VMEM((3, CHUNK, D), jnp.float32),
        pltpu.VMEM_SHARED((N, D), jnp.float32),
    ] + [pltpu.SemaphoreType.DMA] * 9,
    compiler_params=_cp,
)
def _sc_aggregate(xv, xc, mli, mlv, mci, mcv, adjp, adjn, zeros_hbm, out,
                  idx_src, idx_dst, val_v, rows, acc, *sems9):
    isem = sems9[0:3]
    gsem = sems9[3:6]
    ssem = sems9[6:9]
    core = lax.axis_index("c")
    sid = lax.axis_index("s")

    def zero_own():
        @pl.when(sid < 10)
        def _():
            pltpu.sync_copy(zeros_hbm, acc.at[pl.ds(sid * 1000, 1000)])

    def drain_own(slot):
        @pl.when(sid < 10)
        def _():
            b = sid * 1000
            pltpu.sync_copy(acc.at[pl.ds(b, 1000)], out.at[slot, pl.ds(b, 1000)])

    def do_agg(table, src_hbm, dst_hbm, val_hbm, slot, n_edges):
        nchunks = n_edges // CHUNK
        my = nchunks // NS + jnp.where(sid < nchunks % NS, 1, 0)

        def idx_fetch(c, b):
            off = (c * NS + sid) * CHUNK
            pltpu.async_copy(src_hbm.at[pl.ds(off, CHUNK)], idx_src.at[b],
                             isem[b])
            pltpu.async_copy(dst_hbm.at[pl.ds(off, CHUNK)], idx_dst.at[b],
                             isem[b])
            if val_hbm is not None:
                pltpu.async_copy(val_hbm.at[pl.ds(off, CHUNK)], val_v.at[b],
                                 isem[b])

        def idx_wait(b):
            pltpu.make_async_copy(src_hbm.at[pl.ds(0, CHUNK)], idx_src.at[b],
                                  isem[b]).wait()
            pltpu.make_async_copy(dst_hbm.at[pl.ds(0, CHUNK)], idx_dst.at[b],
                                  isem[b]).wait()
            if val_hbm is not None:
                pltpu.make_async_copy(val_hbm.at[pl.ds(0, CHUNK)],
                                      val_v.at[b], isem[b]).wait()

        def gather_issue(b):
            pltpu.async_copy(table.at[idx_src.at[b]], rows.at[b], gsem[b])

        def gather_wait(b):
            pltpu.make_async_copy(table.at[idx_src.at[b]], rows.at[b],
                                  gsem[b]).wait()

        def scatter_issue(b):
            pltpu.async_copy(rows.at[b], acc.at[idx_dst.at[b]], ssem[b],
                             add=True)

        def scatter_wait(b):
            pltpu.make_async_copy(rows.at[b], acc.at[idx_dst.at[b]],
                                  ssem[b]).wait()

        def scale(b):
            if val_hbm is not None:
                @plsc.parallel_loop(0, CHUNK, unroll=4)
                def _(i):
                    vv = plsc.load_gather(val_v.at[b],
                                          [jnp.full((16,), i, jnp.int32)])
                    for j in range(0, D, 16):
                        rows[b, i, pl.ds(j, 16)] = rows[b, i, pl.ds(j, 16)] * vv

        idx_fetch(0, 0)
        idx_fetch(1, 1)
        idx_wait(0)
        gather_issue(0)

        @pl.loop(0, (my + 2) // 3)
        def _(k):
            for r in range(3):
                c = k * 3 + r
                b0, b1, b2 = r % 3, (r + 1) % 3, (r + 2) % 3

                @pl.when(c < my)
                def _(c=c, b0=b0, b1=b1, b2=b2):
                    @pl.when(c >= 1)
                    def _():
                        scatter_wait(b2)

                    @pl.when(c + 2 < my)
                    def _():
                        idx_fetch(c + 2, b2)

                    @pl.when(c + 1 < my)
                    def _():
                        idx_wait(b1)
                        gather_issue(b1)
                    gather_wait(b0)
                    scale(b0)
                    scatter_issue(b0)

        for j in range(3):
            @pl.when((my - 1) % 3 == j)
            def _(j=j):
                scatter_wait(j)

        plsc.subcore_barrier()
        drain_own(slot)
        zero_own()
        plsc.subcore_barrier()

    zero_own()
    plsc.subcore_barrier()

    @pl.when(core == 0)
    def _():
        for p in range(4):
            do_agg(xv, mli.at[p, 1], mli.at[p, 0], mlv.at[p], p, EM)
        do_agg(xc, adjp.at[0], adjp.at[1], None, 4, E)
        do_agg(xc, adjn.at[0], adjn.at[1], None, 5, E)

    @pl.when(core == 1)
    def _():
        for p in range(4):
            do_agg(xc, mci.at[p, 1], mci.at[p, 0], mcv.at[p], 6 + p, EM)
        do_agg(xv, adjp.at[1], adjp.at[0], None, 10, E)
        do_agg(xv, adjn.at[1], adjn.at[0], None, 11, E)


BLK = 1000


def _tc_body(final, xv_ref, xc_ref, aggs_ref, w_ref, g_ref, b_ref,
             hv_ref, hc_ref):
    def side(x_ref, k0, out_ref):
        h = x_ref[...]
        for k in range(k0, k0 + 6):
            h = h + jnp.maximum(
                jnp.dot(aggs_ref[k], w_ref[k],
                        preferred_element_type=jnp.float32), 0.0)
        if final:
            mu = jnp.mean(h, axis=-1, keepdims=True)
            var = jnp.mean((h - mu) ** 2, axis=-1, keepdims=True)
            h = (h - mu) * lax.rsqrt(var + 1e-5) * g_ref[...] + b_ref[...]
        out_ref[...] = h

    side(xv_ref, 0, hv_ref)
    side(xc_ref, 6, hc_ref)


def _tc_update(xv, xc, aggs, w12, gamma2, beta2, final):
    return pl.pallas_call(
        functools.partial(_tc_body, final),
        grid=(N // BLK,),
        in_specs=[
            pl.BlockSpec((BLK, D), lambda i: (i, 0)),
            pl.BlockSpec((BLK, D), lambda i: (i, 0)),
            pl.BlockSpec((12, BLK, D), lambda i: (0, i, 0)),
            pl.BlockSpec((12, D, D), lambda i: (0, 0, 0)),
            pl.BlockSpec((1, D), lambda i: (0, 0)),
            pl.BlockSpec((1, D), lambda i: (0, 0)),
        ],
        out_specs=[
            pl.BlockSpec((BLK, D), lambda i: (i, 0)),
            pl.BlockSpec((BLK, D), lambda i: (i, 0)),
        ],
        out_shape=[jax.ShapeDtypeStruct((N, D), jnp.float32)] * 2,
    )(xv, xc, aggs, w12, gamma2, beta2)


def kernel(xv, xc, adj_pos, adj_neg, meta_lit_idx, meta_lit_val,
           meta_cls_idx, meta_cls_val, W_lit, W_cls, W_cv_pos, W_cv_neg,
           W_vc_pos, W_vc_neg, gamma, beta):
    adjp = adj_pos.astype(jnp.int32)
    adjn = adj_neg.astype(jnp.int32)
    mli = meta_lit_idx.astype(jnp.int32)
    mci = meta_cls_idx.astype(jnp.int32)
    gamma2 = gamma.reshape(1, D)
    beta2 = beta.reshape(1, D)
    zeros_hbm = jnp.zeros((1000, D), jnp.float32)
    for l in range(L):
        aggs = _sc_aggregate(xv, xc, mli, meta_lit_val, mci, meta_cls_val,
                             adjp, adjn, zeros_hbm)
        w12 = jnp.concatenate([
            W_lit[l], W_cv_pos[l][None], W_cv_neg[l][None],
            W_cls[l], W_vc_pos[l][None], W_vc_neg[l][None],
        ], axis=0)
        xv, xc = _tc_update(xv, xc, aggs, w12, gamma2, beta2, final=(l == L - 1))
    return (xv, xc)
